# baseline (speedup 1.0000x reference)
import functools

import jax
import jax.numpy as jnp
from jax import lax
from jax.experimental import pallas as pl
from jax.experimental.pallas import tpu as pltpu
from jax.experimental.pallas import tpu_sc as plsc

B = 4096
L = 50
D = 64
NC, NS = 2, 16
NW = NC * NS
RPW = B // NW
NBUF = 2
CR = 8
NCH = RPW // CR
LANES = 16


def _bag_sums(cat_flat, emb_table):
    mesh = plsc.VectorSubcoreMesh(
        core_axis_name="c", subcore_axis_name="s", num_cores=NC, num_subcores=NS
    )

    @functools.partial(
        pl.kernel,
        mesh=mesh,
        out_type=jax.ShapeDtypeStruct((B, D), jnp.float32),
        scratch_types=[
            pltpu.VMEM((RPW * L,), jnp.int32),
            pltpu.VMEM((NBUF, CR * L, D), jnp.float32),
            pltpu.VMEM((RPW, D), jnp.float32),
            pltpu.SemaphoreType.DMA((NBUF,)),
        ],
        compiler_params=pltpu.CompilerParams(use_tc_tiling_on_sc=False),
    )
    def k(idx_hbm, table_hbm, out_hbm, idx_v, bufs, out_v, sems):
        wid = lax.axis_index("s") * NC + lax.axis_index("c")
        base = wid * RPW
        pltpu.sync_copy(idx_hbm.at[pl.ds(base * L, RPW * L)], idx_v)

        def fire(g, slot):
            pltpu.async_copy(
                table_hbm.at[idx_v.at[pl.ds(g * CR * L, CR * L)]],
                bufs.at[slot],
                sems.at[slot],
            )

        for b in range(NBUF):
            fire(b, b)

        def body(gg, _):
            for b in range(NBUF):
                g = gg * NBUF + b
                pltpu.make_async_copy(
                    table_hbm.at[idx_v.at[pl.ds(g * CR * L, CR * L)]],
                    bufs.at[b],
                    sems.at[b],
                ).wait()
                for r in range(CR):
                    accs = [jnp.zeros((LANES,), jnp.float32) for _ in range(D // LANES)]
                    for j in range(L):
                        for c in range(D // LANES):
                            accs[c] = accs[c] + bufs[b, r * L + j, pl.ds(c * LANES, LANES)]
                    for c in range(D // LANES):
                        out_v[g * CR + r, pl.ds(c * LANES, LANES)] = accs[c]

                @pl.when(g + NBUF < NCH)
                def _():
                    fire(g + NBUF, b)
            return 0

        lax.fori_loop(0, NCH // NBUF, body, 0)
        pltpu.sync_copy(out_v, out_hbm.at[pl.ds(base, RPW)])

    return k(cat_flat, emb_table)


BLK = 512


def _mlp_body(cont_ref, sums_ref, cat_ref, row0_ref,
              w1_ref, b1_ref, w2_ref, b2_ref, w3_ref, b3_ref, out_ref):
    cat = cat_ref[...]
    cnt = jnp.sum((cat != 0).astype(jnp.float32), axis=1)
    n0 = jnp.float32(L) - cnt
    row0 = row0_ref[0, :]
    bag = sums_ref[...] - n0[:, None] * row0[None, :]
    bag = bag / jnp.maximum(cnt, 1.0)[:, None]
    bag = jnp.where(cnt[:, None] > 0, bag, 0.0)
    combined = jnp.concatenate([cont_ref[...], bag], axis=1)
    h = jnp.dot(combined, w1_ref[...], preferred_element_type=jnp.float32)
    h = jnp.maximum(h + b1_ref[0, :][None, :], 0.0)
    h = jnp.dot(h, w2_ref[...], preferred_element_type=jnp.float32)
    h = jnp.maximum(h + b2_ref[0, :][None, :], 0.0)
    h = jnp.dot(h, w3_ref[...], preferred_element_type=jnp.float32)
    out_ref[...] = h + b3_ref[0, :][None, :]


def _mlp(cont, sums, cat, row0, w1, b1, w2, b2, w3, b3):
    grid = (B // BLK,)
    blk = lambda r, c: pl.BlockSpec((r, c), lambda i: (i, 0))
    full = lambda r, c: pl.BlockSpec((r, c), lambda i: (0, 0))
    return pl.pallas_call(
        _mlp_body,
        grid=grid,
        in_specs=[
            blk(BLK, 64),
            blk(BLK, 64),
            blk(BLK, L),
            full(1, 64),
            full(128, 512), full(1, 512),
            full(512, 256), full(1, 256),
            full(256, 128), full(1, 128),
        ],
        out_specs=blk(BLK, 128),
        out_shape=jax.ShapeDtypeStruct((B, 128), jnp.float32),
    )(cont, sums, cat, row0, w1, b1, w2, b2, w3, b3)


@jax.jit
def kernel(continuous, categories, emb_table, W1, b1, W2, b2, W3, b3):
    cat = categories.astype(jnp.int32)
    sums = _bag_sums(cat.reshape(-1), emb_table)
    return _mlp(
        continuous, sums, cat, emb_table[0:1, :],
        W1, b1.reshape(1, -1), W2, b2.reshape(1, -1), W3, b3.reshape(1, -1),
    )


# speedup vs baseline: 1.0101x; 1.0101x over previous
import functools

import jax
import jax.numpy as jnp
from jax import lax
from jax.experimental import pallas as pl
from jax.experimental.pallas import tpu as pltpu
from jax.experimental.pallas import tpu_sc as plsc

B = 4096
L = 50
D = 64
NC, NS = 2, 16
NW = NC * NS
RPW = B // NW
NBUF = 4
CR = 4
NCH = RPW // CR
LANES = 16


def _bag_sums(cat_flat, emb_table):
    mesh = plsc.VectorSubcoreMesh(
        core_axis_name="c", subcore_axis_name="s", num_cores=NC, num_subcores=NS
    )

    @functools.partial(
        pl.kernel,
        mesh=mesh,
        out_type=jax.ShapeDtypeStruct((B, D), jnp.float32),
        scratch_types=[
            pltpu.VMEM((RPW * L,), jnp.int32),
            pltpu.VMEM((NBUF, CR * L, D), jnp.float32),
            pltpu.VMEM((RPW, D), jnp.float32),
            pltpu.SemaphoreType.DMA((NBUF,)),
        ],
        compiler_params=pltpu.CompilerParams(use_tc_tiling_on_sc=False),
    )
    def k(idx_hbm, table_hbm, out_hbm, idx_v, bufs, out_v, sems):
        wid = lax.axis_index("s") * NC + lax.axis_index("c")
        base = wid * RPW
        pltpu.sync_copy(idx_hbm.at[pl.ds(base * L, RPW * L)], idx_v)

        def fire(g, slot):
            pltpu.async_copy(
                table_hbm.at[idx_v.at[pl.ds(g * CR * L, CR * L)]],
                bufs.at[slot],
                sems.at[slot],
            )

        for b in range(NBUF):
            fire(b, b)

        def body(gg, _):
            for b in range(NBUF):
                g = gg * NBUF + b
                pltpu.make_async_copy(
                    table_hbm.at[idx_v.at[pl.ds(g * CR * L, CR * L)]],
                    bufs.at[b],
                    sems.at[b],
                ).wait()
                for r in range(CR):
                    accs = [jnp.zeros((LANES,), jnp.float32) for _ in range(D // LANES)]
                    for j in range(L):
                        for c in range(D // LANES):
                            accs[c] = accs[c] + bufs[b, r * L + j, pl.ds(c * LANES, LANES)]
                    for c in range(D // LANES):
                        out_v[g * CR + r, pl.ds(c * LANES, LANES)] = accs[c]

                @pl.when(g + NBUF < NCH)
                def _():
                    fire(g + NBUF, b)
            return 0

        lax.fori_loop(0, NCH // NBUF, body, 0)
        pltpu.sync_copy(out_v, out_hbm.at[pl.ds(base, RPW)])

    return k(cat_flat, emb_table)


BLK = 512


def _mlp_body(cont_ref, sums_ref, cat_ref, row0_ref,
              w1_ref, b1_ref, w2_ref, b2_ref, w3_ref, b3_ref, out_ref):
    cat = cat_ref[...]
    cnt = jnp.sum((cat != 0).astype(jnp.float32), axis=1)
    n0 = jnp.float32(L) - cnt
    row0 = row0_ref[0, :]
    bag = sums_ref[...] - n0[:, None] * row0[None, :]
    bag = bag / jnp.maximum(cnt, 1.0)[:, None]
    bag = jnp.where(cnt[:, None] > 0, bag, 0.0)
    combined = jnp.concatenate([cont_ref[...], bag], axis=1)
    h = jnp.dot(combined, w1_ref[...], preferred_element_type=jnp.float32)
    h = jnp.maximum(h + b1_ref[0, :][None, :], 0.0)
    h = jnp.dot(h, w2_ref[...], preferred_element_type=jnp.float32)
    h = jnp.maximum(h + b2_ref[0, :][None, :], 0.0)
    h = jnp.dot(h, w3_ref[...], preferred_element_type=jnp.float32)
    out_ref[...] = h + b3_ref[0, :][None, :]


def _mlp(cont, sums, cat, row0, w1, b1, w2, b2, w3, b3):
    grid = (B // BLK,)
    blk = lambda r, c: pl.BlockSpec((r, c), lambda i: (i, 0))
    full = lambda r, c: pl.BlockSpec((r, c), lambda i: (0, 0))
    return pl.pallas_call(
        _mlp_body,
        grid=grid,
        in_specs=[
            blk(BLK, 64),
            blk(BLK, 64),
            blk(BLK, L),
            full(1, 64),
            full(128, 512), full(1, 512),
            full(512, 256), full(1, 256),
            full(256, 128), full(1, 128),
        ],
        out_specs=blk(BLK, 128),
        out_shape=jax.ShapeDtypeStruct((B, 128), jnp.float32),
    )(cont, sums, cat, row0, w1, b1, w2, b2, w3, b3)


@jax.jit
def kernel(continuous, categories, emb_table, W1, b1, W2, b2, W3, b3):
    cat = categories.astype(jnp.int32)
    sums = _bag_sums(cat.reshape(-1), emb_table)
    return _mlp(
        continuous, sums, cat, emb_table[0:1, :],
        W1, b1.reshape(1, -1), W2, b2.reshape(1, -1), W3, b3.reshape(1, -1),
    )


# speedup vs baseline: 1.0676x; 1.0569x over previous
import functools

import jax
import jax.numpy as jnp
from jax import lax
from jax.experimental import pallas as pl
from jax.experimental.pallas import tpu as pltpu
from jax.experimental.pallas import tpu_sc as plsc

B = 4096
L = 50
D = 64
NC, NS = 2, 16
NW = NC * NS
RPW = B // NW
NBUF = 2
CR = 4
NCH = RPW // CR
LANES = 16


def _bag_sums(cat_flat, emb_table):
    mesh = plsc.VectorSubcoreMesh(
        core_axis_name="c", subcore_axis_name="s", num_cores=NC, num_subcores=NS
    )

    @functools.partial(
        pl.kernel,
        mesh=mesh,
        out_type=jax.ShapeDtypeStruct((B, D), jnp.float32),
        scratch_types=[
            pltpu.VMEM((RPW * L,), jnp.int32),
            pltpu.VMEM((NBUF, CR * L, D), jnp.float32),
            pltpu.VMEM((RPW, D), jnp.float32),
            pltpu.SemaphoreType.DMA((NBUF,)),
        ],
        compiler_params=pltpu.CompilerParams(use_tc_tiling_on_sc=False),
    )
    def k(idx_hbm, table_hbm, out_hbm, idx_v, bufs, out_v, sems):
        wid = lax.axis_index("s") * NC + lax.axis_index("c")
        base = wid * RPW
        pltpu.sync_copy(idx_hbm.at[pl.ds(base * L, RPW * L)], idx_v)

        def fire(g, slot):
            pltpu.async_copy(
                table_hbm.at[idx_v.at[pl.ds(g * CR * L, CR * L)]],
                bufs.at[slot],
                sems.at[slot],
            )

        for b in range(NBUF):
            fire(b, b)

        def body(gg, _):
            for b in range(NBUF):
                g = gg * NBUF + b
                pltpu.make_async_copy(
                    table_hbm.at[idx_v.at[pl.ds(g * CR * L, CR * L)]],
                    bufs.at[b],
                    sems.at[b],
                ).wait()
                for r in range(CR):
                    accs = [jnp.zeros((LANES,), jnp.float32) for _ in range(D // LANES)]
                    for j in range(L):
                        for c in range(D // LANES):
                            accs[c] = accs[c] + bufs[b, r * L + j, pl.ds(c * LANES, LANES)]
                    for c in range(D // LANES):
                        out_v[g * CR + r, pl.ds(c * LANES, LANES)] = accs[c]

                @pl.when(g + NBUF < NCH)
                def _():
                    fire(g + NBUF, b)
            return 0

        lax.fori_loop(0, NCH // NBUF, body, 0)
        pltpu.sync_copy(out_v, out_hbm.at[pl.ds(base, RPW)])

    return k(cat_flat, emb_table)


BLK = 512


def _mlp_body(cont_ref, sums_ref, cat_ref, row0_ref,
              w1_ref, b1_ref, w2_ref, b2_ref, w3_ref, b3_ref, out_ref):
    cat = cat_ref[...]
    cnt = jnp.sum((cat != 0).astype(jnp.float32), axis=1)
    n0 = jnp.float32(L) - cnt
    row0 = row0_ref[0, :]
    bag = sums_ref[...] - n0[:, None] * row0[None, :]
    bag = bag / jnp.maximum(cnt, 1.0)[:, None]
    bag = jnp.where(cnt[:, None] > 0, bag, 0.0)
    combined = jnp.concatenate([cont_ref[...], bag], axis=1)
    h = jnp.dot(combined, w1_ref[...], preferred_element_type=jnp.float32)
    h = jnp.maximum(h + b1_ref[0, :][None, :], 0.0)
    h = jnp.dot(h, w2_ref[...], preferred_element_type=jnp.float32)
    h = jnp.maximum(h + b2_ref[0, :][None, :], 0.0)
    h = jnp.dot(h, w3_ref[...], preferred_element_type=jnp.float32)
    out_ref[...] = h + b3_ref[0, :][None, :]


def _mlp(cont, sums, cat, row0, w1, b1, w2, b2, w3, b3):
    grid = (B // BLK,)
    blk = lambda r, c: pl.BlockSpec((r, c), lambda i: (i, 0))
    full = lambda r, c: pl.BlockSpec((r, c), lambda i: (0, 0))
    return pl.pallas_call(
        _mlp_body,
        grid=grid,
        in_specs=[
            blk(BLK, 64),
            blk(BLK, 64),
            blk(BLK, L),
            full(1, 64),
            full(128, 512), full(1, 512),
            full(512, 256), full(1, 256),
            full(256, 128), full(1, 128),
        ],
        out_specs=blk(BLK, 128),
        out_shape=jax.ShapeDtypeStruct((B, 128), jnp.float32),
    )(cont, sums, cat, row0, w1, b1, w2, b2, w3, b3)


@jax.jit
def kernel(continuous, categories, emb_table, W1, b1, W2, b2, W3, b3):
    cat = categories.astype(jnp.int32)
    sums = _bag_sums(cat.reshape(-1), emb_table)
    return _mlp(
        continuous, sums, cat, emb_table[0:1, :],
        W1, b1.reshape(1, -1), W2, b2.reshape(1, -1), W3, b3.reshape(1, -1),
    )


# speedup vs baseline: 1.0765x; 1.0083x over previous
import functools

import jax
import jax.numpy as jnp
from jax import lax
from jax.experimental import pallas as pl
from jax.experimental.pallas import tpu as pltpu
from jax.experimental.pallas import tpu_sc as plsc

B = 4096
L = 50
D = 64
NC, NS = 2, 16
NW = NC * NS
RPW = B // NW
NBUF = 2
CR = 4
NCH = RPW // CR
LANES = 16


def _bag_sums(cat, emb_table):
    mesh = plsc.VectorSubcoreMesh(
        core_axis_name="c", subcore_axis_name="s", num_cores=NC, num_subcores=NS
    )

    @functools.partial(
        pl.kernel,
        mesh=mesh,
        out_type=jax.ShapeDtypeStruct((B, D), jnp.float32),
        scratch_types=[
            pltpu.VMEM((RPW, L), jnp.int32),
            pltpu.VMEM((RPW * L,), jnp.int32),
            pltpu.VMEM((NBUF, CR * L, D), jnp.float32),
            pltpu.VMEM((RPW, D), jnp.float32),
            pltpu.SemaphoreType.DMA((NBUF,)),
        ],
        compiler_params=pltpu.CompilerParams(use_tc_tiling_on_sc=False),
    )
    def k(idx_hbm, table_hbm, out_hbm, idx2d, idx_v, bufs, out_v, sems):
        wid = lax.axis_index("s") * NC + lax.axis_index("c")
        base = wid * RPW
        pltpu.sync_copy(idx_hbm.at[pl.ds(base, RPW)], idx2d)

        def repack(r, _):
            for off in (0, 16, 32, L - LANES):
                idx_v[pl.ds(r * L + off, LANES)] = idx2d[r, pl.ds(off, LANES)]
            return 0

        lax.fori_loop(0, RPW, repack, 0)

        def fire(g, slot):
            pltpu.async_copy(
                table_hbm.at[idx_v.at[pl.ds(g * CR * L, CR * L)]],
                bufs.at[slot],
                sems.at[slot],
            )

        for b in range(NBUF):
            fire(b, b)

        def body(gg, _):
            for b in range(NBUF):
                g = gg * NBUF + b
                pltpu.make_async_copy(
                    table_hbm.at[idx_v.at[pl.ds(g * CR * L, CR * L)]],
                    bufs.at[b],
                    sems.at[b],
                ).wait()
                for r in range(CR):
                    accs = [jnp.zeros((LANES,), jnp.float32) for _ in range(D // LANES)]
                    for j in range(L):
                        for c in range(D // LANES):
                            accs[c] = accs[c] + bufs[b, r * L + j, pl.ds(c * LANES, LANES)]
                    for c in range(D // LANES):
                        out_v[g * CR + r, pl.ds(c * LANES, LANES)] = accs[c]

                @pl.when(g + NBUF < NCH)
                def _():
                    fire(g + NBUF, b)
            return 0

        lax.fori_loop(0, NCH // NBUF, body, 0)
        pltpu.sync_copy(out_v, out_hbm.at[pl.ds(base, RPW)])

    return k(cat, emb_table)


BLK = 512


def _mlp_body(cont_ref, sums_ref, cat_ref, row0_ref,
              w1_ref, b1_ref, w2_ref, b2_ref, w3_ref, b3_ref, out_ref):
    cat = cat_ref[...]
    cnt = jnp.sum((cat != 0).astype(jnp.float32), axis=1)
    n0 = jnp.float32(L) - cnt
    row0 = row0_ref[0, :]
    bag = sums_ref[...] - n0[:, None] * row0[None, :]
    bag = bag / jnp.maximum(cnt, 1.0)[:, None]
    bag = jnp.where(cnt[:, None] > 0, bag, 0.0)
    combined = jnp.concatenate([cont_ref[...], bag], axis=1)
    h = jnp.dot(combined, w1_ref[...], preferred_element_type=jnp.float32)
    h = jnp.maximum(h + b1_ref[0, :][None, :], 0.0)
    h = jnp.dot(h, w2_ref[...], preferred_element_type=jnp.float32)
    h = jnp.maximum(h + b2_ref[0, :][None, :], 0.0)
    h = jnp.dot(h, w3_ref[...], preferred_element_type=jnp.float32)
    out_ref[...] = h + b3_ref[0, :][None, :]


def _mlp(cont, sums, cat, row0, w1, b1, w2, b2, w3, b3):
    grid = (B // BLK,)
    blk = lambda r, c: pl.BlockSpec((r, c), lambda i: (i, 0))
    full = lambda r, c: pl.BlockSpec((r, c), lambda i: (0, 0))
    return pl.pallas_call(
        _mlp_body,
        grid=grid,
        in_specs=[
            blk(BLK, 64),
            blk(BLK, 64),
            blk(BLK, L),
            full(1, 64),
            full(128, 512), full(1, 512),
            full(512, 256), full(1, 256),
            full(256, 128), full(1, 128),
        ],
        out_specs=blk(BLK, 128),
        out_shape=jax.ShapeDtypeStruct((B, 128), jnp.float32),
    )(cont, sums, cat, row0, w1, b1, w2, b2, w3, b3)


@jax.jit
def kernel(continuous, categories, emb_table, W1, b1, W2, b2, W3, b3):
    cat = categories.astype(jnp.int32)
    sums = _bag_sums(cat, emb_table)
    return _mlp(
        continuous, sums, cat, emb_table[0:1, :],
        W1, b1.reshape(1, -1), W2, b2.reshape(1, -1), W3, b3.reshape(1, -1),
    )


# speedup vs baseline: 1.0852x; 1.0081x over previous
import functools

import jax
import jax.numpy as jnp
from jax import lax
from jax.experimental import pallas as pl
from jax.experimental.pallas import tpu as pltpu
from jax.experimental.pallas import tpu_sc as plsc

B = 4096
L = 50
D = 64
NC, NS = 2, 16
NW = NC * NS
RPW = B // NW
NBUF = 2
CR = 4
NCH = RPW // CR
LANES = 16


def _bag_sums(cat, emb_table):
    mesh = plsc.VectorSubcoreMesh(
        core_axis_name="c", subcore_axis_name="s", num_cores=NC, num_subcores=NS
    )

    @functools.partial(
        pl.kernel,
        mesh=mesh,
        out_type=jax.ShapeDtypeStruct((B, D), jnp.float32),
        scratch_types=[
            pltpu.VMEM((RPW, 64), jnp.int32),
            pltpu.VMEM((RPW * L,), jnp.int32),
            pltpu.VMEM((NBUF, CR * L, D), jnp.float32),
            pltpu.VMEM((RPW, D), jnp.float32),
            pltpu.SemaphoreType.DMA((NBUF,)),
        ],
        compiler_params=pltpu.CompilerParams(use_tc_tiling_on_sc=False),
    )
    def k(idx_hbm, table_hbm, out_hbm, idx2d, idx_v, bufs, out_v, sems):
        wid = lax.axis_index("s") * NC + lax.axis_index("c")
        base = wid * RPW
        pltpu.sync_copy(idx_hbm.at[pl.ds(base, RPW)], idx2d)

        def repack(r, _):
            for off in (0, 16, 32, L - LANES):
                idx_v[pl.ds(r * L + off, LANES)] = idx2d[r, pl.ds(off, LANES)]
            return 0

        lax.fori_loop(0, RPW, repack, 0)

        def fire(g, slot):
            pltpu.async_copy(
                table_hbm.at[idx_v.at[pl.ds(g * CR * L, CR * L)]],
                bufs.at[slot],
                sems.at[slot],
            )

        for b in range(NBUF):
            fire(b, b)

        def body(gg, _):
            for b in range(NBUF):
                g = gg * NBUF + b
                pltpu.make_async_copy(
                    table_hbm.at[idx_v.at[pl.ds(g * CR * L, CR * L)]],
                    bufs.at[b],
                    sems.at[b],
                ).wait()
                for r in range(CR):
                    accs = [jnp.zeros((LANES,), jnp.float32) for _ in range(D // LANES)]
                    for j in range(L):
                        for c in range(D // LANES):
                            accs[c] = accs[c] + bufs[b, r * L + j, pl.ds(c * LANES, LANES)]
                    for c in range(D // LANES):
                        out_v[g * CR + r, pl.ds(c * LANES, LANES)] = accs[c]

                @pl.when(g + NBUF < NCH)
                def _():
                    fire(g + NBUF, b)
            return 0

        lax.fori_loop(0, NCH // NBUF, body, 0)
        pltpu.sync_copy(out_v, out_hbm.at[pl.ds(base, RPW)])

    return k(cat, emb_table)


BLK = 1024


def _mlp_body(cont_ref, sums_ref, cat_ref, row0_ref,
              w1_ref, b1_ref, w2_ref, b2_ref, w3_ref, b3_ref, out_ref):
    cat = cat_ref[...]
    cnt = jnp.sum((cat != 0).astype(jnp.float32), axis=1)
    n0 = jnp.float32(L) - cnt
    row0 = row0_ref[0, :]
    bag = sums_ref[...] - n0[:, None] * row0[None, :]
    bag = bag / jnp.maximum(cnt, 1.0)[:, None]
    bag = jnp.where(cnt[:, None] > 0, bag, 0.0)
    combined = jnp.concatenate([cont_ref[...], bag], axis=1)
    h = jnp.dot(combined, w1_ref[...], preferred_element_type=jnp.float32)
    h = jnp.maximum(h + b1_ref[0, :][None, :], 0.0)
    h = jnp.dot(h, w2_ref[...], preferred_element_type=jnp.float32)
    h = jnp.maximum(h + b2_ref[0, :][None, :], 0.0)
    h = jnp.dot(h, w3_ref[...], preferred_element_type=jnp.float32)
    out_ref[...] = h + b3_ref[0, :][None, :]


def _mlp(cont, sums, cat, row0, w1, b1, w2, b2, w3, b3):
    grid = (B // BLK,)
    blk = lambda r, c: pl.BlockSpec((r, c), lambda i: (i, 0))
    full = lambda r, c: pl.BlockSpec((r, c), lambda i: (0, 0))
    return pl.pallas_call(
        _mlp_body,
        grid=grid,
        in_specs=[
            blk(BLK, 64),
            blk(BLK, 64),
            blk(BLK, L),
            full(1, 64),
            full(128, 512), full(1, 512),
            full(512, 256), full(1, 256),
            full(256, 128), full(1, 128),
        ],
        out_specs=blk(BLK, 128),
        out_shape=jax.ShapeDtypeStruct((B, 128), jnp.float32),
    )(cont, sums, cat, row0, w1, b1, w2, b2, w3, b3)


@jax.jit
def kernel(continuous, categories, emb_table, W1, b1, W2, b2, W3, b3):
    cat = categories.astype(jnp.int32)
    cat64 = jnp.pad(cat, ((0, 0), (0, 64 - L)))
    sums = _bag_sums(cat64, emb_table)
    return _mlp(
        continuous, sums, cat, emb_table[0:1, :],
        W1, b1.reshape(1, -1), W2, b2.reshape(1, -1), W3, b3.reshape(1, -1),
    )


# speedup vs baseline: 1.0982x; 1.0119x over previous
import functools

import jax
import jax.numpy as jnp
from jax import lax
from jax.experimental import pallas as pl
from jax.experimental.pallas import tpu as pltpu
from jax.experimental.pallas import tpu_sc as plsc

B = 4096
L = 50
D = 64
NC, NS = 2, 16
NW = NC * NS
RPW = B // NW
NBUF = 2
CR = 4
NCH = RPW // CR
LANES = 16


def _bag_sums(cat, emb_table):
    mesh = plsc.VectorSubcoreMesh(
        core_axis_name="c", subcore_axis_name="s", num_cores=NC, num_subcores=NS
    )

    @functools.partial(
        pl.kernel,
        mesh=mesh,
        out_type=jax.ShapeDtypeStruct((B, D), jnp.float32),
        scratch_types=[
            pltpu.VMEM((RPW, 64), jnp.float32),
            pltpu.VMEM((RPW * L,), jnp.int32),
            pltpu.VMEM((NBUF, CR * L, D), jnp.float32),
            pltpu.VMEM((RPW, D), jnp.float32),
            pltpu.SemaphoreType.DMA((NBUF,)),
        ],
        compiler_params=pltpu.CompilerParams(use_tc_tiling_on_sc=False, needs_layout_passes=False),
    )
    def k(idx_hbm, table_hbm, out_hbm, idx2d, idx_v, bufs, out_v, sems):
        wid = lax.axis_index("s") * NC + lax.axis_index("c")
        base = wid * RPW
        pltpu.sync_copy(idx_hbm.at[pl.ds(base, RPW)], idx2d)

        def repack(r, _):
            for off in (0, 16, 32, L - LANES):
                idx_v[pl.ds(r * L + off, LANES)] = plsc.bitcast(
                    idx2d[r, pl.ds(off, LANES)], jnp.int32)
            return 0

        lax.fori_loop(0, RPW, repack, 0)

        def fire(g, slot):
            pltpu.async_copy(
                table_hbm.at[idx_v.at[pl.ds(g * CR * L, CR * L)]],
                bufs.at[slot],
                sems.at[slot],
            )

        for b in range(NBUF):
            fire(b, b)

        def body(gg, _):
            for b in range(NBUF):
                g = gg * NBUF + b
                pltpu.make_async_copy(
                    table_hbm.at[idx_v.at[pl.ds(g * CR * L, CR * L)]],
                    bufs.at[b],
                    sems.at[b],
                ).wait()
                for r in range(CR):
                    accs = [jnp.zeros((LANES,), jnp.float32) for _ in range(D // LANES)]
                    for j in range(L):
                        for c in range(D // LANES):
                            accs[c] = accs[c] + bufs[b, r * L + j, pl.ds(c * LANES, LANES)]
                    for c in range(D // LANES):
                        out_v[g * CR + r, pl.ds(c * LANES, LANES)] = accs[c]

                @pl.when(g + NBUF < NCH)
                def _():
                    fire(g + NBUF, b)
            return 0

        lax.fori_loop(0, NCH // NBUF, body, 0)
        pltpu.sync_copy(out_v, out_hbm.at[pl.ds(base, RPW)])

    return k(cat, emb_table)


BLK = 1024


def _mlp_body(cont_ref, sums_ref, cat_ref, row0_ref,
              w1_ref, b1_ref, w2_ref, b2_ref, w3_ref, b3_ref, out_ref):
    cat = cat_ref[...]
    cnt = jnp.sum((cat != 0).astype(jnp.float32), axis=1)
    n0 = jnp.float32(L) - cnt
    row0 = row0_ref[0, :]
    bag = sums_ref[...] - n0[:, None] * row0[None, :]
    bag = bag / jnp.maximum(cnt, 1.0)[:, None]
    bag = jnp.where(cnt[:, None] > 0, bag, 0.0)
    combined = jnp.concatenate([cont_ref[...], bag], axis=1)
    h = jnp.dot(combined, w1_ref[...], preferred_element_type=jnp.float32)
    h = jnp.maximum(h + b1_ref[0, :][None, :], 0.0)
    h = jnp.dot(h, w2_ref[...], preferred_element_type=jnp.float32)
    h = jnp.maximum(h + b2_ref[0, :][None, :], 0.0)
    h = jnp.dot(h, w3_ref[...], preferred_element_type=jnp.float32)
    out_ref[...] = h + b3_ref[0, :][None, :]


def _mlp(cont, sums, cat, row0, w1, b1, w2, b2, w3, b3):
    grid = (B // BLK,)
    blk = lambda r, c: pl.BlockSpec((r, c), lambda i: (i, 0))
    full = lambda r, c: pl.BlockSpec((r, c), lambda i: (0, 0))
    return pl.pallas_call(
        _mlp_body,
        grid=grid,
        in_specs=[
            blk(BLK, 64),
            blk(BLK, 64),
            blk(BLK, L),
            full(1, 64),
            full(128, 512), full(1, 512),
            full(512, 256), full(1, 256),
            full(256, 128), full(1, 128),
        ],
        out_specs=blk(BLK, 128),
        out_shape=jax.ShapeDtypeStruct((B, 128), jnp.float32),
    )(cont, sums, cat, row0, w1, b1, w2, b2, w3, b3)


@jax.jit
def kernel(continuous, categories, emb_table, W1, b1, W2, b2, W3, b3):
    cat = categories.astype(jnp.int32)
    cat64 = jnp.pad(cat, ((0, 0), (0, 64 - L)))
    catf = lax.bitcast_convert_type(cat64, jnp.float32)
    sums = _bag_sums(catf, emb_table)
    return _mlp(
        continuous, sums, cat, emb_table[0:1, :],
        W1, b1.reshape(1, -1), W2, b2.reshape(1, -1), W3, b3.reshape(1, -1),
    )
